# baseline (speedup 1.0000x reference)
import jax
import jax.numpy as jnp
from jax.experimental import pallas as pl

_B, _J, _D = 1024, 66, 240
_NUP = 42
_BBLK = 32


_RDJ = 48


def _body(x_ref, out_ref, masked_ref):
    x = x_ref[...][:, :_NUP, :]
    out_ref[...] = x
    masked_ref[...] = jnp.concatenate(
        [x, jnp.zeros((x.shape[0], _J - _NUP, _D), x.dtype)], axis=1
    )


def kernel(input):
    out, masked = pl.pallas_call(
        _body,
        grid=(_B // _BBLK,),
        in_specs=[pl.BlockSpec((_BBLK, _RDJ, _D), lambda i: (i, 0, 0))],
        out_specs=[
            pl.BlockSpec((_BBLK, _NUP, _D), lambda i: (i, 0, 0)),
            pl.BlockSpec((_BBLK, _J, _D), lambda i: (i, 0, 0)),
        ],
        out_shape=[
            jax.ShapeDtypeStruct((_B, _NUP, _D), input.dtype),
            jax.ShapeDtypeStruct((_B, _J, _D), input.dtype),
        ],
    )(input)
    return (out, masked)


# speedup vs baseline: 5.0569x; 5.0569x over previous
import jax
import jax.numpy as jnp
from jax.experimental import pallas as pl

_B, _J, _D = 1024, 66, 240
_NUP = 42
_BBLK = 128


def _body(x_ref, out_ref, masked_ref):
    x = x_ref[...]
    out_ref[...] = x
    masked_ref[:_NUP] = x
    masked_ref[_NUP:] = jnp.zeros((_J - _NUP, _D, x.shape[2]), x.dtype)


def kernel(input):
    xt = jnp.transpose(input, (1, 2, 0))
    out_t, masked_t = pl.pallas_call(
        _body,
        grid=(_B // _BBLK,),
        in_specs=[pl.BlockSpec((_NUP, _D, _BBLK), lambda i: (0, 0, i))],
        out_specs=[
            pl.BlockSpec((_NUP, _D, _BBLK), lambda i: (0, 0, i)),
            pl.BlockSpec((_J, _D, _BBLK), lambda i: (0, 0, i)),
        ],
        out_shape=[
            jax.ShapeDtypeStruct((_NUP, _D, _B), input.dtype),
            jax.ShapeDtypeStruct((_J, _D, _B), input.dtype),
        ],
    )(xt)
    return (jnp.transpose(out_t, (2, 0, 1)), jnp.transpose(masked_t, (2, 0, 1)))
